# Initial kernel scaffold; baseline (speedup 1.0000x reference)
import functools

import jax
import jax.numpy as jnp
from jax import lax
from jax.experimental import pallas as pl
from jax.experimental.pallas import tpu as pltpu
from jax.experimental.pallas import tpu_sc as plsc

N = 10000
E = 320000
D = 128
ROW = 144
INV_SCALE = 0.25

NC = 2
NS = 16
NW = NC * NS
EPW = E // NW
C = 80
NCHUNK = EPW // C
GROUPS = C // 16
RPW = N // NS
RBLK = 125

MBLK = 400


def _qkv_body(dst_ref, src_ref, wq_ref, wk_ref, wv_ref, bq_ref, bk_ref, bv_ref,
              q_ref, k_ref, v_ref):
    dn = (((1,), (1,)), ((), ()))
    q_ref[...] = lax.dot_general(dst_ref[...], wq_ref[...], dn,
                                 preferred_element_type=jnp.float32) + bq_ref[...]
    k_ref[...] = lax.dot_general(src_ref[...], wk_ref[...], dn,
                                 preferred_element_type=jnp.float32) + bk_ref[...]
    v_ref[...] = lax.dot_general(src_ref[...], wv_ref[...], dn,
                                 preferred_element_type=jnp.float32) + bv_ref[...]


def _qkv(dst_feat, src_feat, Wq, Wk, Wv, bq, bk, bv):
    grid = (N // MBLK,)
    row_spec = pl.BlockSpec((MBLK, D), lambda i: (i, 0))
    w_spec = pl.BlockSpec((D, D), lambda i: (0, 0))
    b_spec = pl.BlockSpec((1, D), lambda i: (0, 0))
    out = jax.ShapeDtypeStruct((N, D), jnp.float32)
    return pl.pallas_call(
        _qkv_body,
        grid=grid,
        in_specs=[row_spec, row_spec, w_spec, w_spec, w_spec,
                  b_spec, b_spec, b_spec],
        out_specs=[row_spec, row_spec, row_spec],
        out_shape=[out, out, out],
    )(dst_feat, src_feat, Wq, Wk, Wv, bq[None], bk[None], bv[None])


def _edge_body(src_hbm, dst_hbm, q_hbm, k_hbm, v_hbm, out_hbm,
               idxs, idxd, qrows, krows, vrows, wv, pmat, zbuf, acc_sh,
               sem_q, sem_k, sem_v):
    cid = lax.axis_index("c")
    sid = lax.axis_index("s")
    wid = cid * NS + sid

    zero16 = jnp.zeros((16,), jnp.float32)
    lane = lax.iota(jnp.int32, 16)

    def zrow(r, carry):
        for j in range(ROW // 16):
            zbuf[r, pl.ds(j * 16, 16)] = zero16
        return carry
    lax.fori_loop(0, RBLK, zrow, 0)
    for i in range(RPW // RBLK):
        pltpu.sync_copy(zbuf, acc_sh.at[pl.ds(sid * RPW + i * RBLK, RBLK)])
    plsc.subcore_barrier()

    ebase = wid * EPW

    def chunk(c, carry):
        off = ebase + c * C
        pltpu.sync_copy(src_hbm.at[pl.ds(off, C)], idxs)
        pltpu.sync_copy(dst_hbm.at[pl.ds(off, C)], idxd)
        cp_q = pltpu.async_copy(q_hbm.at[idxd], qrows, sem_q)
        cp_k = pltpu.async_copy(k_hbm.at[idxs], krows, sem_k)
        cp_v = pltpu.async_copy(v_hbm.at[idxs], vrows, sem_v)
        cp_q.wait()
        cp_k.wait()
        cp_v.wait()

        def group(g, gcarry):
            r0 = g * 16
            for e in range(16):
                r = r0 + e
                acc = qrows[r, pl.ds(0, 16)] * krows[r, pl.ds(0, 16)]
                for j in range(1, D // 16):
                    acc = acc + (qrows[r, pl.ds(j * 16, 16)] *
                                 krows[r, pl.ds(j * 16, 16)])
                pmat[e, :] = acc
            s = plsc.load_gather(pmat, [lane, jnp.full((16,), 0, jnp.int32)])
            for l in range(1, 16):
                s = s + plsc.load_gather(pmat, [lane, jnp.full((16,), l, jnp.int32)])
            w = jnp.exp(s * INV_SCALE)
            for e in range(16):
                r = r0 + e
                w_e = w[e]
                for j in range(D // 16):
                    wv[r, pl.ds(j * 16, 16)] = vrows[r, pl.ds(j * 16, 16)] * w_e
                wv[r, pl.ds(D, 16)] = jnp.where(lane == e, w_e, 0.0)
            return gcarry
        lax.fori_loop(0, GROUPS, group, 0)

        pltpu.sync_copy(wv, acc_sh.at[idxd], add=True)
        return carry
    lax.fori_loop(0, NCHUNK, chunk, 0)

    plsc.subcore_barrier()

    for i in range(RPW // RBLK):
        rs = sid * RPW + i * RBLK
        pltpu.sync_copy(acc_sh.at[pl.ds(rs, RBLK)], zbuf)
        pltpu.sync_copy(zbuf, out_hbm.at[cid, pl.ds(rs, RBLK)])


_edge_kernel = functools.partial(
    pl.kernel,
    out_type=jax.ShapeDtypeStruct((NC, N, ROW), jnp.float32),
    mesh=plsc.VectorSubcoreMesh(core_axis_name="c", subcore_axis_name="s"),
    scratch_types=[
        pltpu.VMEM((C,), jnp.int32),
        pltpu.VMEM((C,), jnp.int32),
        pltpu.VMEM((C, D), jnp.float32),
        pltpu.VMEM((C, D), jnp.float32),
        pltpu.VMEM((C, D), jnp.float32),
        pltpu.VMEM((C, ROW), jnp.float32),
        pltpu.VMEM((16, 16), jnp.float32),
        pltpu.VMEM((RBLK, ROW), jnp.float32),
        pltpu.VMEM_SHARED((N, ROW), jnp.float32),
        pltpu.SemaphoreType.DMA,
        pltpu.SemaphoreType.DMA,
        pltpu.SemaphoreType.DMA,
    ],
)(_edge_body)


def _combine_body(p_ref, o_ref):
    s = p_ref[0] + p_ref[1]
    num = s[:, :D]
    den = jnp.sum(s[:, D:], axis=1, keepdims=True)
    o_ref[...] = num / jnp.maximum(den, 1e-9)


def _combine(parts):
    return pl.pallas_call(
        _combine_body,
        grid=(N // MBLK,),
        in_specs=[pl.BlockSpec((NC, MBLK, ROW), lambda i: (0, i, 0))],
        out_specs=pl.BlockSpec((MBLK, D), lambda i: (i, 0)),
        out_shape=jax.ShapeDtypeStruct((N, D), jnp.float32),
    )(parts)


def kernel(src_feat, dst_feat, edge_index, Wq, bq, Wk, bk, Wv, bv):
    q, k, v = _qkv(dst_feat, src_feat, Wq, Wk, Wv, bq, bk, bv)
    src = edge_index[0]
    dst = edge_index[1]
    parts = _edge_kernel(src, dst, q, k, v)
    return _combine(parts)


# speedup vs baseline: 7.0205x; 7.0205x over previous
import functools

import jax
import jax.numpy as jnp
from jax import lax
from jax.experimental import pallas as pl
from jax.experimental.pallas import tpu as pltpu
from jax.experimental.pallas import tpu_sc as plsc

N = 10000
E = 320000
D = 128
ROW = 144
INV_SCALE = 0.25

NC = 2
NS = 16
NW = NC * NS
EPW = E // NW
C = 80
NCHUNK = EPW // C
GROUPS = C // 16
NACC = 10240
RPW = NACC // NS
RBLK = 128

MBLK = 400


def _qkv_body(dst_ref, src_ref, wq_ref, wk_ref, wv_ref, bq_ref, bk_ref, bv_ref,
              q_ref, k_ref, v_ref):
    dn = (((1,), (1,)), ((), ()))
    q_ref[...] = (lax.dot_general(dst_ref[...], wq_ref[...], dn,
                                  preferred_element_type=jnp.float32)
                  + bq_ref[...]).astype(jnp.bfloat16)
    k_ref[...] = (lax.dot_general(src_ref[...], wk_ref[...], dn,
                                  preferred_element_type=jnp.float32)
                  + bk_ref[...]).astype(jnp.bfloat16)
    v_ref[...] = lax.dot_general(src_ref[...], wv_ref[...], dn,
                                 preferred_element_type=jnp.float32) + bv_ref[...]


def _qkv(dst_feat, src_feat, Wq, Wk, Wv, bq, bk, bv):
    grid = (N // MBLK,)
    row_spec = pl.BlockSpec((MBLK, D), lambda i: (i, 0))
    w_spec = pl.BlockSpec((D, D), lambda i: (0, 0))
    b_spec = pl.BlockSpec((1, D), lambda i: (0, 0))
    out_bf = jax.ShapeDtypeStruct((N, D), jnp.bfloat16)
    out_f32 = jax.ShapeDtypeStruct((N, D), jnp.float32)
    return pl.pallas_call(
        _qkv_body,
        grid=grid,
        in_specs=[row_spec, row_spec, w_spec, w_spec, w_spec,
                  b_spec, b_spec, b_spec],
        out_specs=[row_spec, row_spec, row_spec],
        out_shape=[out_bf, out_bf, out_f32],
    )(dst_feat, src_feat, Wq, Wk, Wv, bq[None], bk[None], bv[None])


def _edge_body(src_hbm, dst_hbm, q_hbm, k_hbm, v_hbm, out_hbm,
               idxs, idxd, qrows, krows, vrows, wv, acc_sh,
               sem_q, sem_k, sem_v):
    cid = lax.axis_index("c")
    sid = lax.axis_index("s")
    wid = cid * NS + sid

    zero16 = jnp.zeros((16,), jnp.float32)
    lane = lax.iota(jnp.int32, 16)

    def zrow(r, carry):
        for j in range(ROW // 16):
            wv[r, pl.ds(j * 16, 16)] = zero16
        return carry
    lax.fori_loop(0, C, zrow, 0)
    for i in range(RPW // C):
        pltpu.sync_copy(wv, acc_sh.at[pl.ds(sid * RPW + i * C, C)])
    plsc.subcore_barrier()

    ebase = wid * EPW

    def chunk(c, carry):
        off = ebase + c * C
        pltpu.sync_copy(src_hbm.at[pl.ds(off, C)], idxs)
        pltpu.sync_copy(dst_hbm.at[pl.ds(off, C)], idxd)
        cp_q = pltpu.async_copy(q_hbm.at[idxd], qrows, sem_q)
        cp_k = pltpu.async_copy(k_hbm.at[idxs], krows, sem_k)
        cp_v = pltpu.async_copy(v_hbm.at[idxs], vrows, sem_v)
        cp_q.wait()
        cp_k.wait()
        cp_v.wait()

        def group(g, gcarry):
            r0 = g * 16
            for e in range(16):
                r = r0 + e
                qb = qrows[r, pl.ds(0, 32)]
                kb = krows[r, pl.ds(0, 32)]
                ph, plo = plsc.unpack(qb * kb, format=plsc.PackFormat.INTERLEAVED)
                acc = ph + plo
                for j in range(1, D // 32):
                    qb = qrows[r, pl.ds(j * 32, 32)]
                    kb = krows[r, pl.ds(j * 32, 32)]
                    ph, plo = plsc.unpack(qb * kb,
                                          format=plsc.PackFormat.INTERLEAVED)
                    acc = acc + ph + plo
                s_e = jnp.sum(acc) * INV_SCALE
                w_vec = jnp.exp(jnp.full((16,), s_e, jnp.float32))
                for j in range(D // 16):
                    wv[r, pl.ds(j * 16, 16)] = vrows[r, pl.ds(j * 16, 16)] * w_vec
                wv[r, pl.ds(D, 16)] = jnp.where(lane == e, w_vec, 0.0)
            return gcarry
        lax.fori_loop(0, GROUPS, group, 0)

        pltpu.sync_copy(wv, acc_sh.at[idxd], add=True)
        return carry
    lax.fori_loop(0, NCHUNK, chunk, 0)

    plsc.subcore_barrier()

    for i in range(RPW // C):
        rs = sid * RPW + i * C
        pltpu.sync_copy(acc_sh.at[pl.ds(rs, C)], wv)
        pltpu.sync_copy(wv, out_hbm.at[cid, pl.ds(rs, C)])


_edge_kernel = functools.partial(
    pl.kernel,
    out_type=jax.ShapeDtypeStruct((NC, NACC, ROW), jnp.float32),
    mesh=plsc.VectorSubcoreMesh(core_axis_name="c", subcore_axis_name="s"),
    compiler_params=pltpu.CompilerParams(needs_layout_passes=False,
                                         use_tc_tiling_on_sc=False),
    scratch_types=[
        pltpu.VMEM((C,), jnp.int32),
        pltpu.VMEM((C,), jnp.int32),
        pltpu.VMEM((C, D), jnp.bfloat16),
        pltpu.VMEM((C, D), jnp.bfloat16),
        pltpu.VMEM((C, D), jnp.float32),
        pltpu.VMEM((C, ROW), jnp.float32),
        pltpu.VMEM_SHARED((NACC, ROW), jnp.float32),
        pltpu.SemaphoreType.DMA,
        pltpu.SemaphoreType.DMA,
        pltpu.SemaphoreType.DMA,
    ],
)(_edge_body)


def _combine_body(p_ref, o_ref):
    s = p_ref[0] + p_ref[1]
    num = s[:, :D]
    den = jnp.sum(s[:, D:], axis=1, keepdims=True)
    o_ref[...] = num / jnp.maximum(den, 1e-9)


def _combine(parts):
    return pl.pallas_call(
        _combine_body,
        grid=(N // MBLK,),
        in_specs=[pl.BlockSpec((NC, MBLK, ROW), lambda i: (0, i, 0))],
        out_specs=pl.BlockSpec((MBLK, D), lambda i: (i, 0)),
        out_shape=jax.ShapeDtypeStruct((N, D), jnp.float32),
    )(parts)


def kernel(src_feat, dst_feat, edge_index, Wq, bq, Wk, bk, Wv, bv):
    q, k, v = _qkv(dst_feat, src_feat, Wq, Wk, Wv, bq, bk, bv)
    src = edge_index[0]
    dst = edge_index[1]
    parts = _edge_kernel(src, dst, q, k, v)
    return _combine(parts)


# speedup vs baseline: 8.4759x; 1.2073x over previous
import functools

import jax
import jax.numpy as jnp
from jax import lax
from jax.experimental import pallas as pl
from jax.experimental.pallas import tpu as pltpu
from jax.experimental.pallas import tpu_sc as plsc

N = 10000
E = 320000
D = 128
ROW = 144
INV_SCALE = 0.25

NC = 2
NS = 16
NW = NC * NS
EPW = E // NW
C = 48
NCHUNK = EPW // C
TAIL = EPW - NCHUNK * C
GROUPS = C // 16
NACC = 10240
RPW = NACC // NS
ZBLK = 40

MBLK = 400


def _qkv_body(dst_ref, src_ref, wq_ref, wk_ref, wv_ref, bq_ref, bk_ref, bv_ref,
              q_ref, k_ref, v_ref):
    dn = (((1,), (1,)), ((), ()))
    q_ref[...] = (lax.dot_general(dst_ref[...], wq_ref[...], dn,
                                  preferred_element_type=jnp.float32)
                  + bq_ref[...]).astype(jnp.bfloat16)
    k_ref[...] = (lax.dot_general(src_ref[...], wk_ref[...], dn,
                                  preferred_element_type=jnp.float32)
                  + bk_ref[...]).astype(jnp.bfloat16)
    v_ref[...] = lax.dot_general(src_ref[...], wv_ref[...], dn,
                                 preferred_element_type=jnp.float32) + bv_ref[...]


def _qkv(dst_feat, src_feat, Wq, Wk, Wv, bq, bk, bv):
    grid = (N // MBLK,)
    row_spec = pl.BlockSpec((MBLK, D), lambda i: (i, 0))
    w_spec = pl.BlockSpec((D, D), lambda i: (0, 0))
    b_spec = pl.BlockSpec((1, D), lambda i: (0, 0))
    out_bf = jax.ShapeDtypeStruct((N, D), jnp.bfloat16)
    out_f32 = jax.ShapeDtypeStruct((N, D), jnp.float32)
    return pl.pallas_call(
        _qkv_body,
        grid=grid,
        in_specs=[row_spec, row_spec, w_spec, w_spec, w_spec,
                  b_spec, b_spec, b_spec],
        out_specs=[row_spec, row_spec, row_spec],
        out_shape=[out_bf, out_bf, out_f32],
    )(dst_feat, src_feat, Wq, Wk, Wv, bq[None], bk[None], bv[None])


def _edge_body(src_hbm, dst_hbm, q_hbm, k_hbm, v_hbm, out_hbm,
               idxs0, idxs1, idxd0, idxd1, idxsc0, idxsc1, idxts, idxtd,
               q0, q1, k0, k1, v0, v1, wv0, wv1, acc_sh,
               sem_g0, sem_g1, sem_s0, sem_s1, sem_i0, sem_i1):
    idxs = (idxs0, idxs1)
    idxd = (idxd0, idxd1)
    idxsc = (idxsc0, idxsc1)
    qr = (q0, q1)
    kr = (k0, k1)
    vr = (v0, v1)
    wv = (wv0, wv1)
    sem_g = (sem_g0, sem_g1)
    sem_s = (sem_s0, sem_s1)
    sem_i = (sem_i0, sem_i1)

    cid = lax.axis_index("c")
    sid = lax.axis_index("s")
    wid = cid * NS + sid
    ebase = wid * EPW
    zero16 = jnp.zeros((16,), jnp.float32)
    izero16 = jnp.zeros((16,), jnp.int32)
    lane = lax.iota(jnp.int32, 16)

    def start_idx(b, c):
        off = jnp.minimum(ebase + c * C, E - C)
        pltpu.make_async_copy(src_hbm.at[pl.ds(off, C)], idxs[b], sem_i[b]).start()
        pltpu.make_async_copy(dst_hbm.at[pl.ds(off, C)], idxd[b], sem_i[b]).start()

    def wait_idx(b):
        pltpu.make_async_copy(src_hbm.at[pl.ds(0, C)], idxs[b], sem_i[b]).wait()
        pltpu.make_async_copy(dst_hbm.at[pl.ds(0, C)], idxd[b], sem_i[b]).wait()

    def start_gathers(b):
        pltpu.make_async_copy(q_hbm.at[idxd[b]], qr[b], sem_g[b]).start()
        pltpu.make_async_copy(k_hbm.at[idxs[b]], kr[b], sem_g[b]).start()
        pltpu.make_async_copy(v_hbm.at[idxs[b]], vr[b], sem_g[b]).start()

    def wait_gathers(b):
        pltpu.make_async_copy(q_hbm.at[idxd[b]], qr[b], sem_g[b]).wait()
        pltpu.make_async_copy(k_hbm.at[idxs[b]], kr[b], sem_g[b]).wait()
        pltpu.make_async_copy(v_hbm.at[idxs[b]], vr[b], sem_g[b]).wait()

    def wait_scatter(b):
        pltpu.make_async_copy(wv[b], acc_sh.at[idxsc[b]], sem_s[b]).wait()

    def do_group(qref, kref, vref, wref, r0):
        for e in range(16):
            r = r0 + e
            qv = qref[r, pl.ds(0, 32)]
            kv = kref[r, pl.ds(0, 32)]
            ph, pl_ = plsc.unpack(qv * kv, format=plsc.PackFormat.INTERLEAVED)
            acc = ph + pl_
            for j in range(1, D // 32):
                qv = qref[r, pl.ds(j * 32, 32)]
                kv = kref[r, pl.ds(j * 32, 32)]
                ph, pl_ = plsc.unpack(qv * kv,
                                      format=plsc.PackFormat.INTERLEAVED)
                acc = acc + ph + pl_
            s_e = jnp.sum(acc) * INV_SCALE
            w_vec = jnp.exp(jnp.full((16,), s_e, jnp.float32))
            for j in range(D // 16):
                wref[r, pl.ds(j * 16, 16)] = vref[r, pl.ds(j * 16, 16)] * w_vec
            wref[r, pl.ds(D, 16)] = jnp.where(lane == e, w_vec, 0.0)

    for b in range(2):
        def zrow(r, carry, _b=b):
            for j in range(ROW // 16):
                wv[_b][r, pl.ds(j * 16, 16)] = zero16
            return carry
        lax.fori_loop(0, C, zrow, 0)
        for t in range(C // 16):
            idxsc[b][pl.ds(t * 16, 16)] = izero16
    for i in range(RPW // ZBLK):
        pltpu.sync_copy(wv0.at[pl.ds(0, ZBLK)],
                        acc_sh.at[pl.ds(sid * RPW + i * ZBLK, ZBLK)])
    plsc.subcore_barrier()

    pltpu.async_copy(wv0, acc_sh.at[idxsc0], sem_s0, add=True)
    pltpu.async_copy(wv1, acc_sh.at[idxsc1], sem_s1, add=True)
    start_idx(0, 0)
    start_idx(1, 1)
    wait_idx(0)
    start_gathers(0)

    def pair(pp, carry):
        for b in range(2):
            c = 2 * pp + b
            nb = 1 - b
            wait_idx(nb)
            start_gathers(nb)
            wait_gathers(b)
            wait_scatter(b)

            def group(g, gcarry, _b=b):
                do_group(qr[_b], kr[_b], vr[_b], wv[_b], g * 16)
                return gcarry
            lax.fori_loop(0, GROUPS, group, 0)
            for t in range(C // 16):
                idxsc[b][pl.ds(t * 16, 16)] = idxd[b][pl.ds(t * 16, 16)]
            pltpu.async_copy(wv[b], acc_sh.at[idxsc[b]], sem_s[b], add=True)
            start_idx(b, c + 2)
        return carry
    lax.fori_loop(0, NCHUNK // 2, pair, 0)

    wait_idx(1)
    wait_gathers(0)
    wait_scatter(0)
    wait_scatter(1)

    toff = ebase + NCHUNK * C
    pltpu.sync_copy(src_hbm.at[pl.ds(toff, TAIL)], idxts)
    pltpu.sync_copy(dst_hbm.at[pl.ds(toff, TAIL)], idxtd)
    pltpu.async_copy(q_hbm.at[idxtd], q0.at[pl.ds(0, TAIL)], sem_g0)
    pltpu.async_copy(k_hbm.at[idxts], k0.at[pl.ds(0, TAIL)], sem_g0)
    pltpu.async_copy(v_hbm.at[idxts], v0.at[pl.ds(0, TAIL)], sem_g0)
    pltpu.make_async_copy(q_hbm.at[idxtd], q0.at[pl.ds(0, TAIL)], sem_g0).wait()
    pltpu.make_async_copy(k_hbm.at[idxts], k0.at[pl.ds(0, TAIL)], sem_g0).wait()
    pltpu.make_async_copy(v_hbm.at[idxts], v0.at[pl.ds(0, TAIL)], sem_g0).wait()
    do_group(q0, k0, v0, wv0, 0)
    pltpu.async_copy(wv0.at[pl.ds(0, TAIL)], acc_sh.at[idxtd], sem_s0, add=True)
    pltpu.make_async_copy(wv0.at[pl.ds(0, TAIL)], acc_sh.at[idxtd], sem_s0).wait()

    plsc.subcore_barrier()

    for i in range(RPW // ZBLK):
        rs = sid * RPW + i * ZBLK
        pltpu.sync_copy(acc_sh.at[pl.ds(rs, ZBLK)], wv0.at[pl.ds(0, ZBLK)])
        pltpu.sync_copy(wv0.at[pl.ds(0, ZBLK)], out_hbm.at[cid, pl.ds(rs, ZBLK)])


_edge_kernel = functools.partial(
    pl.kernel,
    out_type=jax.ShapeDtypeStruct((NC, NACC, ROW), jnp.float32),
    mesh=plsc.VectorSubcoreMesh(core_axis_name="c", subcore_axis_name="s"),
    compiler_params=pltpu.CompilerParams(needs_layout_passes=False,
                                         use_tc_tiling_on_sc=False),
    scratch_types=[
        pltpu.VMEM((C,), jnp.int32),
        pltpu.VMEM((C,), jnp.int32),
        pltpu.VMEM((C,), jnp.int32),
        pltpu.VMEM((C,), jnp.int32),
        pltpu.VMEM((C,), jnp.int32),
        pltpu.VMEM((C,), jnp.int32),
        pltpu.VMEM((TAIL,), jnp.int32),
        pltpu.VMEM((TAIL,), jnp.int32),
        pltpu.VMEM((C, D), jnp.bfloat16),
        pltpu.VMEM((C, D), jnp.bfloat16),
        pltpu.VMEM((C, D), jnp.bfloat16),
        pltpu.VMEM((C, D), jnp.bfloat16),
        pltpu.VMEM((C, D), jnp.float32),
        pltpu.VMEM((C, D), jnp.float32),
        pltpu.VMEM((C, ROW), jnp.float32),
        pltpu.VMEM((C, ROW), jnp.float32),
        pltpu.VMEM_SHARED((NACC, ROW), jnp.float32),
        pltpu.SemaphoreType.DMA,
        pltpu.SemaphoreType.DMA,
        pltpu.SemaphoreType.DMA,
        pltpu.SemaphoreType.DMA,
        pltpu.SemaphoreType.DMA,
        pltpu.SemaphoreType.DMA,
    ],
)(_edge_body)


def _combine_body(p_ref, o_ref):
    s = p_ref[0] + p_ref[1]
    num = s[:, :D]
    den = jnp.sum(s[:, D:], axis=1, keepdims=True)
    o_ref[...] = num / jnp.maximum(den, 1e-9)


def _combine(parts):
    return pl.pallas_call(
        _combine_body,
        grid=(N // MBLK,),
        in_specs=[pl.BlockSpec((NC, MBLK, ROW), lambda i: (0, i, 0))],
        out_specs=pl.BlockSpec((MBLK, D), lambda i: (i, 0)),
        out_shape=jax.ShapeDtypeStruct((N, D), jnp.float32),
    )(parts)


def kernel(src_feat, dst_feat, edge_index, Wq, bq, Wk, bk, Wv, bv):
    q, k, v = _qkv(dst_feat, src_feat, Wq, Wk, Wv, bq, bk, bv)
    src = edge_index[0]
    dst = edge_index[1]
    parts = _edge_kernel(src, dst, q, k, v)
    return _combine(parts)
